# baseline (speedup 1.0000x reference)
import functools

import jax
import jax.numpy as jnp
from jax import lax
from jax.experimental import pallas as pl
from jax.experimental.pallas import tpu as pltpu
from jax.experimental.pallas import tpu_sc as plsc

_VOCAB = 100000
_EMBED = 128
_TRAIN = 10000
_FIXED = _VOCAB - _TRAIN

_LANES = 16
_NC = 2
_NS = 16
_NW = _NC * _NS

_K = 128

_CROWS = 2000


def _concat_tables(fixed3, train3):
    nf = fixed3.shape[0]
    nt = train3.shape[0]
    grid = nf + nt
    rows = fixed3.shape[1]

    def body(f_ref, t_ref, o_ref):
        i = pl.program_id(0)

        @pl.when(i < nf)
        def _():
            o_ref[...] = f_ref[0]

        @pl.when(i >= nf)
        def _():
            o_ref[...] = t_ref[0]

    return pl.pallas_call(
        body,
        grid=(grid,),
        in_specs=[
            pl.BlockSpec((1, rows, _EMBED), lambda i: (jnp.minimum(i, nf - 1), 0, 0)),
            pl.BlockSpec((1, rows, _EMBED), lambda i: (jnp.maximum(i - nf, 0), 0, 0)),
        ],
        out_specs=pl.BlockSpec((rows, _EMBED), lambda i: (i, 0)),
        out_shape=jax.ShapeDtypeStruct((grid * rows, _EMBED), jnp.float32),
    )(fixed3, train3)


_NBUF = 4
_DEPTH = 2

_TPAD = 64


_TROW = 56


def _sc_gather(table, ids3, s0, s1):
    n_chunks = ids3.shape[1]
    b_per_w = s0 // _NW
    mesh = plsc.VectorSubcoreMesh(core_axis_name="c", subcore_axis_name="s")

    @functools.partial(
        pl.kernel,
        mesh=mesh,
        out_type=jax.ShapeDtypeStruct((s0 * _TROW, _EMBED), jnp.float32),
        scratch_types=[pltpu.VMEM((n_chunks, _K), jnp.int32)]
        + [pltpu.VMEM((_K, _EMBED), jnp.float32) for _ in range(_NBUF)]
        + [pltpu.SemaphoreType.DMA for _ in range(_NBUF)],
    )
    def k(table_hbm, ids_hbm, out_hbm, idx_all, *rest):
        bufs = rest[:_NBUF]
        sems = rest[_NBUF:]
        wid = lax.axis_index("s") * _NC + lax.axis_index("c")
        bbase = wid * b_per_w

        pltpu.sync_copy(ids_hbm.at[wid], idx_all)

        def clip_row(r, carry):
            for j in range(_K // _LANES):
                sl = pl.ds(j * _LANES, _LANES)
                idx_all.at[r][sl] = jnp.clip(idx_all.at[r][sl], 0, _VOCAB - 1)
            return carry

        lax.fori_loop(0, n_chunks, clip_row, 0)

        def fire(c, buf, sem):
            pltpu.async_copy(table_hbm.at[idx_all.at[c]], buf, sem)

        def drain(c, buf, sem):
            pltpu.make_async_copy(table_hbm.at[idx_all.at[c]], buf, sem).wait()

        for t in range(_DEPTH):
            fire(t, bufs[t], sems[t])

        def group(m, carry):
            for b in range(_NBUF):
                c = m * _NBUF + b
                cn = jnp.minimum(c + _DEPTH, n_chunks - 1)
                bn = (b + _DEPTH) % _NBUF
                drain(c, bufs[b], sems[b])
                fire(cn, bufs[bn], sems[bn])
                b0 = bbase + 2 * c
                pltpu.sync_copy(
                    bufs[b].at[pl.ds(0, _TROW)],
                    out_hbm.at[pl.ds(b0 * _TROW, _TROW)],
                )
                pltpu.sync_copy(
                    bufs[b].at[pl.ds(_TPAD, _TROW)],
                    out_hbm.at[pl.ds((b0 + 1) * _TROW, _TROW)],
                )
            return carry

        lax.fori_loop(0, n_chunks // _NBUF, group, 0)

        for t in range(_DEPTH):
            b = (n_chunks + t) % _NBUF
            drain(n_chunks - 1, bufs[b], sems[b])

    return k(table, ids3)


def kernel(input_ids, fixed_embedding, train_embedding):
    s0, s1 = input_ids.shape
    fixed3 = fixed_embedding.reshape(_FIXED // _CROWS, _CROWS, _EMBED)
    train3 = train_embedding.reshape(_TRAIN // _CROWS, _CROWS, _EMBED)
    table = _concat_tables(fixed3, train3)
    ids_p = jnp.pad(input_ids, ((0, 0), (0, _TPAD - s1)))
    ids3 = ids_p.reshape(_NW, (s0 * _TPAD) // (_NW * _K), _K)
    out_flat = _sc_gather(table, ids3, s0, s1)
    return out_flat.reshape(s0, _TROW, _EMBED)[:, :s1, :]


# speedup vs baseline: 11.7207x; 11.7207x over previous
import functools

import jax
import jax.numpy as jnp
from jax import lax
from jax.experimental import pallas as pl
from jax.experimental.pallas import tpu as pltpu
from jax.experimental.pallas import tpu_sc as plsc

_VOCAB = 100000
_EMBED = 128
_TRAIN = 10000
_FIXED = _VOCAB - _TRAIN

_LANES = 16
_NC = 2
_NS = 16
_NW = _NC * _NS

_K = 128

_CROWS = 2000


def _concat_tables(fixed3, train3):
    nf = fixed3.shape[0]
    nt = train3.shape[0]
    grid = nf + nt
    rows = fixed3.shape[1]

    def body(f_ref, t_ref, o_ref):
        i = pl.program_id(0)

        @pl.when(i < nf)
        def _():
            o_ref[...] = f_ref[0]

        @pl.when(i >= nf)
        def _():
            o_ref[...] = t_ref[0]

    return pl.pallas_call(
        body,
        grid=(grid,),
        in_specs=[
            pl.BlockSpec((1, rows, _EMBED), lambda i: (jnp.minimum(i, nf - 1), 0, 0)),
            pl.BlockSpec((1, rows, _EMBED), lambda i: (jnp.maximum(i - nf, 0), 0, 0)),
        ],
        out_specs=pl.BlockSpec((rows, _EMBED), lambda i: (i, 0)),
        out_shape=jax.ShapeDtypeStruct((grid * rows, _EMBED), jnp.float32),
    )(fixed3, train3)


_NBUF = 4
_DEPTH = 2

_TPAD = 64


def _sc_gather(table, ids3, s0, s1):
    n_chunks = ids3.shape[1]
    b_per_w = s0 // _NW
    mesh = plsc.VectorSubcoreMesh(core_axis_name="c", subcore_axis_name="s")

    @functools.partial(
        pl.kernel,
        mesh=mesh,
        out_type=jax.ShapeDtypeStruct((s0, s1, _EMBED), jnp.float32),
        scratch_types=[pltpu.VMEM((n_chunks, _K), jnp.int32)]
        + [pltpu.VMEM((_K, _EMBED), jnp.float32) for _ in range(_NBUF)]
        + [pltpu.SemaphoreType.DMA for _ in range(_NBUF)],
        compiler_params=pltpu.CompilerParams(use_tc_tiling_on_sc=True),
    )
    def k(table_hbm, ids_hbm, out_hbm, idx_all, *rest):
        bufs = rest[:_NBUF]
        sems = rest[_NBUF:]
        wid = lax.axis_index("s") * _NC + lax.axis_index("c")
        bbase = wid * b_per_w

        pltpu.sync_copy(ids_hbm.at[wid], idx_all)

        def clip_row(r, carry):
            for j in range(_K // _LANES):
                sl = pl.ds(j * _LANES, _LANES)
                idx_all.at[r][sl] = jnp.clip(idx_all.at[r][sl], 0, _VOCAB - 1)
            return carry

        lax.fori_loop(0, n_chunks, clip_row, 0)

        def fire(c, buf, sem):
            pltpu.async_copy(table_hbm.at[idx_all.at[c]], buf, sem)

        def drain(c, buf, sem):
            pltpu.make_async_copy(table_hbm.at[idx_all.at[c]], buf, sem).wait()

        for t in range(_DEPTH):
            fire(t, bufs[t], sems[t])

        def group(m, carry):
            for b in range(_NBUF):
                c = m * _NBUF + b
                cn = jnp.minimum(c + _DEPTH, n_chunks - 1)
                bn = (b + _DEPTH) % _NBUF
                drain(c, bufs[b], sems[b])
                fire(cn, bufs[bn], sems[bn])
                b0 = bbase + 2 * c
                pltpu.sync_copy(bufs[b].at[pl.ds(0, s1)], out_hbm.at[b0])
                pltpu.sync_copy(bufs[b].at[pl.ds(_TPAD, s1)], out_hbm.at[b0 + 1])
            return carry

        lax.fori_loop(0, n_chunks // _NBUF, group, 0)

        for t in range(_DEPTH):
            b = (n_chunks + t) % _NBUF
            drain(n_chunks - 1, bufs[b], sems[b])

    return k(table, ids3)


def kernel(input_ids, fixed_embedding, train_embedding):
    s0, s1 = input_ids.shape
    fixed3 = fixed_embedding.reshape(_FIXED // _CROWS, _CROWS, _EMBED)
    train3 = train_embedding.reshape(_TRAIN // _CROWS, _CROWS, _EMBED)
    table = _concat_tables(fixed3, train3)
    npad = _TPAD - s1
    junk = (jnp.arange(s0 * npad, dtype=jnp.int32) % _VOCAB).reshape(s0, npad)
    ids_p = jnp.concatenate([input_ids, junk], axis=1)
    ids3 = ids_p.reshape(_NW, (s0 * _TPAD) // (_NW * _K), _K)
    return _sc_gather(table, ids3, s0, s1)


# speedup vs baseline: 17.9115x; 1.5282x over previous
import functools

import jax
import jax.numpy as jnp
from jax import lax
from jax.experimental import pallas as pl
from jax.experimental.pallas import tpu as pltpu
from jax.experimental.pallas import tpu_sc as plsc

_VOCAB = 100000
_EMBED = 128
_TRAIN = 10000
_FIXED = _VOCAB - _TRAIN

_LANES = 16
_NC = 2
_NS = 16
_NW = _NC * _NS

_K = 128

_CROWS = 2000


def _concat_tables(fixed3, train3):
    nf = fixed3.shape[0]
    nt = train3.shape[0]
    grid = nf + nt
    rows = fixed3.shape[1]

    def body(f_ref, t_ref, o_ref):
        i = pl.program_id(0)

        @pl.when(i < nf)
        def _():
            o_ref[...] = f_ref[0]

        @pl.when(i >= nf)
        def _():
            o_ref[...] = t_ref[0]

    return pl.pallas_call(
        body,
        grid=(grid,),
        in_specs=[
            pl.BlockSpec((1, rows, _EMBED), lambda i: (jnp.minimum(i, nf - 1), 0, 0)),
            pl.BlockSpec((1, rows, _EMBED), lambda i: (jnp.maximum(i - nf, 0), 0, 0)),
        ],
        out_specs=pl.BlockSpec((rows, _EMBED), lambda i: (i, 0)),
        out_shape=jax.ShapeDtypeStruct((grid * rows, _EMBED), jnp.float32),
    )(fixed3, train3)


_NBUF = 5
_DEPTH = 3

_TPAD = 64


def _sc_gather(table, ids_flat):
    batch = ids_flat.shape[0]
    rows_per_w = batch // _NW
    n_chunks = rows_per_w // _K
    mesh = plsc.VectorSubcoreMesh(core_axis_name="c", subcore_axis_name="s")

    @functools.partial(
        pl.kernel,
        mesh=mesh,
        out_type=jax.ShapeDtypeStruct((batch // _K, _K, _EMBED), jnp.float32),
        scratch_types=[
            pltpu.VMEM((rows_per_w,), jnp.int32),
            pltpu.VMEM((n_chunks, _K), jnp.int32),
        ]
        + [pltpu.VMEM((_K, _EMBED), jnp.float32) for _ in range(_NBUF)]
        + [pltpu.SemaphoreType.DMA for _ in range(_NBUF)],
        compiler_params=pltpu.CompilerParams(use_tc_tiling_on_sc=True),
    )
    def k(table_hbm, ids_hbm, out_hbm, idx_raw, idx2, *rest):
        bufs = rest[:_NBUF]
        sems = rest[_NBUF:]
        wid = lax.axis_index("s") * _NC + lax.axis_index("c")
        cbase = wid * n_chunks

        pltpu.sync_copy(ids_hbm.at[pl.ds(wid * rows_per_w, rows_per_w)], idx_raw)

        def clip_pack(g, carry):
            r = g // (_K // _LANES)
            j = g % (_K // _LANES)
            v = idx_raw[pl.ds(g * _LANES, _LANES)]
            idx2.at[r][pl.ds(j * _LANES, _LANES)] = jnp.clip(v, 0, _VOCAB - 1)
            return carry

        lax.fori_loop(0, rows_per_w // _LANES, clip_pack, 0)

        def fire(c, buf, sem):
            pltpu.async_copy(table_hbm.at[idx2.at[c]], buf, sem)

        def drain(c, buf, sem):
            pltpu.make_async_copy(table_hbm.at[idx2.at[c]], buf, sem).wait()

        for t in range(_DEPTH):
            fire(t, bufs[t], sems[t])

        def group(m, carry):
            for b in range(_NBUF):
                c = m * _NBUF + b
                cn = jnp.minimum(c + _DEPTH, n_chunks - 1)
                bn = (b + _DEPTH) % _NBUF
                drain(c, bufs[b], sems[b])
                fire(cn, bufs[bn], sems[bn])
                pltpu.sync_copy(bufs[b], out_hbm.at[cbase + c])
            return carry

        lax.fori_loop(0, n_chunks // _NBUF, group, 0)

        for t in range(_DEPTH):
            b = (n_chunks + t) % _NBUF
            drain(n_chunks - 1, bufs[b], sems[b])

    return k(table, ids_flat)


def kernel(input_ids, fixed_embedding, train_embedding):
    s0, s1 = input_ids.shape
    fixed3 = fixed_embedding.reshape(_FIXED // _CROWS, _CROWS, _EMBED)
    train3 = train_embedding.reshape(_TRAIN // _CROWS, _CROWS, _EMBED)
    table = _concat_tables(fixed3, train3)
    ids_t = input_ids.T.reshape(-1)
    out3 = _sc_gather(table, ids_t)
    return out3.reshape(s1, s0, _EMBED).transpose(1, 0, 2)


# speedup vs baseline: 19.9184x; 1.1120x over previous
import functools

import jax
import jax.numpy as jnp
from jax import lax
from jax.experimental import pallas as pl
from jax.experimental.pallas import tpu as pltpu
from jax.experimental.pallas import tpu_sc as plsc

_VOCAB = 100000
_EMBED = 128
_TRAIN = 10000
_FIXED = _VOCAB - _TRAIN

_LANES = 16
_NC = 2
_NS = 16
_NW = _NC * _NS

_K = 128

_CROWS = 5000


def _concat_tables(fixed3, train3):
    nf = fixed3.shape[0]
    nt = train3.shape[0]
    grid = nf + nt
    rows = fixed3.shape[1]

    def body(f_ref, t_ref, o_ref):
        i = pl.program_id(0)

        @pl.when(i < nf)
        def _():
            o_ref[...] = f_ref[0]

        @pl.when(i >= nf)
        def _():
            o_ref[...] = t_ref[0]

    return pl.pallas_call(
        body,
        grid=(grid,),
        in_specs=[
            pl.BlockSpec((1, rows, _EMBED), lambda i: (jnp.minimum(i, nf - 1), 0, 0)),
            pl.BlockSpec((1, rows, _EMBED), lambda i: (jnp.maximum(i - nf, 0), 0, 0)),
        ],
        out_specs=pl.BlockSpec((rows, _EMBED), lambda i: (i, 0)),
        out_shape=jax.ShapeDtypeStruct((grid * rows, _EMBED), jnp.float32),
    )(fixed3, train3)


_NBUF = 5
_DEPTH = 4

_TPAD = 64


def _sc_gather(table, ids_flat):
    batch = ids_flat.shape[0]
    rows_per_w = batch // _NW
    n_chunks = rows_per_w // _K
    mesh = plsc.VectorSubcoreMesh(core_axis_name="c", subcore_axis_name="s")

    @functools.partial(
        pl.kernel,
        mesh=mesh,
        out_type=jax.ShapeDtypeStruct((batch // _K, _K, _EMBED), jnp.float32),
        scratch_types=[
            pltpu.VMEM((rows_per_w,), jnp.int32),
            pltpu.VMEM((n_chunks, _K), jnp.int32),
        ]
        + [pltpu.VMEM((_K, _EMBED), jnp.float32) for _ in range(_NBUF)]
        + [pltpu.SemaphoreType.DMA for _ in range(_NBUF)],
        compiler_params=pltpu.CompilerParams(use_tc_tiling_on_sc=True),
    )
    def k(table_hbm, ids_hbm, out_hbm, idx_raw, idx2, *rest):
        bufs = rest[:_NBUF]
        sems = rest[_NBUF:]
        wid = lax.axis_index("s") * _NC + lax.axis_index("c")
        cbase = wid * n_chunks

        pltpu.sync_copy(ids_hbm.at[pl.ds(wid * rows_per_w, rows_per_w)], idx_raw)

        def clip_pack(g, carry):
            r = g // (_K // _LANES)
            j = g % (_K // _LANES)
            v = idx_raw[pl.ds(g * _LANES, _LANES)]
            idx2.at[r][pl.ds(j * _LANES, _LANES)] = jnp.clip(v, 0, _VOCAB - 1)
            return carry

        lax.fori_loop(0, rows_per_w // _LANES, clip_pack, 0)

        def fire(c, buf, sem):
            pltpu.async_copy(table_hbm.at[idx2.at[c]], buf, sem)

        def drain(c, buf, sem):
            pltpu.make_async_copy(table_hbm.at[idx2.at[c]], buf, sem).wait()

        for t in range(_DEPTH):
            fire(t, bufs[t], sems[t])

        def group(m, carry):
            for b in range(_NBUF):
                c = m * _NBUF + b
                cn = jnp.minimum(c + _DEPTH, n_chunks - 1)
                bn = (b + _DEPTH) % _NBUF
                drain(c, bufs[b], sems[b])
                fire(cn, bufs[bn], sems[bn])
                pltpu.sync_copy(bufs[b], out_hbm.at[cbase + c])
            return carry

        lax.fori_loop(0, n_chunks // _NBUF, group, 0)

        for t in range(_DEPTH):
            b = (n_chunks + t) % _NBUF
            drain(n_chunks - 1, bufs[b], sems[b])

    return k(table, ids_flat)


def kernel(input_ids, fixed_embedding, train_embedding):
    s0, s1 = input_ids.shape
    fixed3 = fixed_embedding.reshape(_FIXED // _CROWS, _CROWS, _EMBED)
    train3 = train_embedding.reshape(_TRAIN // _CROWS, _CROWS, _EMBED)
    table = _concat_tables(fixed3, train3)
    ids_t = input_ids.T.reshape(-1)
    out3 = _sc_gather(table, ids_t)
    return out3.reshape(s1, s0, _EMBED).transpose(1, 0, 2)


# speedup vs baseline: 20.3750x; 1.0229x over previous
import functools

import jax
import jax.numpy as jnp
from jax import lax
from jax.experimental import pallas as pl
from jax.experimental.pallas import tpu as pltpu
from jax.experimental.pallas import tpu_sc as plsc

_VOCAB = 100000
_EMBED = 128
_TRAIN = 10000
_FIXED = _VOCAB - _TRAIN

_LANES = 16
_NC = 2
_NS = 16
_NW = _NC * _NS

_K = 128

_CROWS = 10000


def _concat_tables(fixed3, train3):
    nf = fixed3.shape[0]
    nt = train3.shape[0]
    grid = nf + nt
    rows = fixed3.shape[1]

    def body(f_ref, t_ref, o_ref):
        i = pl.program_id(0)

        @pl.when(i < nf)
        def _():
            o_ref[...] = f_ref[0]

        @pl.when(i >= nf)
        def _():
            o_ref[...] = t_ref[0]

    return pl.pallas_call(
        body,
        grid=(grid,),
        in_specs=[
            pl.BlockSpec((1, rows, _EMBED), lambda i: (jnp.minimum(i, nf - 1), 0, 0)),
            pl.BlockSpec((1, rows, _EMBED), lambda i: (jnp.maximum(i - nf, 0), 0, 0)),
        ],
        out_specs=pl.BlockSpec((rows, _EMBED), lambda i: (i, 0)),
        out_shape=jax.ShapeDtypeStruct((grid * rows, _EMBED), jnp.float32),
    )(fixed3, train3)


_NBUF = 5
_DEPTH = 4

_TPAD = 64


def _sc_gather(table, ids_flat):
    batch = ids_flat.shape[0]
    rows_per_w = batch // _NW
    n_chunks = rows_per_w // _K
    mesh = plsc.VectorSubcoreMesh(core_axis_name="c", subcore_axis_name="s")

    @functools.partial(
        pl.kernel,
        mesh=mesh,
        out_type=jax.ShapeDtypeStruct((batch // _K, _K, _EMBED), jnp.float32),
        scratch_types=[
            pltpu.VMEM((rows_per_w,), jnp.int32),
            pltpu.VMEM((n_chunks, _K), jnp.int32),
        ]
        + [pltpu.VMEM((_K, _EMBED), jnp.float32) for _ in range(_NBUF)]
        + [pltpu.SemaphoreType.DMA for _ in range(_NBUF)],
        compiler_params=pltpu.CompilerParams(use_tc_tiling_on_sc=True),
    )
    def k(table_hbm, ids_hbm, out_hbm, idx_raw, idx2, *rest):
        bufs = rest[:_NBUF]
        sems = rest[_NBUF:]
        wid = lax.axis_index("s") * _NC + lax.axis_index("c")
        cbase = wid * n_chunks

        pltpu.sync_copy(ids_hbm.at[pl.ds(wid * rows_per_w, rows_per_w)], idx_raw)

        def clip_pack(g, carry):
            r = g // (_K // _LANES)
            j = g % (_K // _LANES)
            v = idx_raw[pl.ds(g * _LANES, _LANES)]
            idx2.at[r][pl.ds(j * _LANES, _LANES)] = jnp.clip(v, 0, _VOCAB - 1)
            return carry

        lax.fori_loop(0, rows_per_w // _LANES, clip_pack, 0)

        def fire(c, buf, sem):
            pltpu.async_copy(table_hbm.at[idx2.at[c]], buf, sem)

        def drain(c, buf, sem):
            pltpu.make_async_copy(table_hbm.at[idx2.at[c]], buf, sem).wait()

        for t in range(_DEPTH):
            fire(t, bufs[t], sems[t])

        def group(m, carry):
            for b in range(_NBUF):
                c = m * _NBUF + b
                cn = jnp.minimum(c + _DEPTH, n_chunks - 1)
                bn = (b + _DEPTH) % _NBUF
                drain(c, bufs[b], sems[b])
                fire(cn, bufs[bn], sems[bn])
                pltpu.sync_copy(bufs[b], out_hbm.at[cbase + c])
            return carry

        lax.fori_loop(0, n_chunks // _NBUF, group, 0)

        for t in range(_DEPTH):
            b = (n_chunks + t) % _NBUF
            drain(n_chunks - 1, bufs[b], sems[b])

    return k(table, ids_flat)


def kernel(input_ids, fixed_embedding, train_embedding):
    s0, s1 = input_ids.shape
    fixed3 = fixed_embedding.reshape(_FIXED // _CROWS, _CROWS, _EMBED)
    train3 = train_embedding.reshape(_TRAIN // _CROWS, _CROWS, _EMBED)
    table = _concat_tables(fixed3, train3)
    ids_t = input_ids.T.reshape(-1)
    out3 = _sc_gather(table, ids_t)
    return out3.reshape(s1, s0, _EMBED).transpose(1, 0, 2)
